# baseline (speedup 1.0000x reference)
import functools

import jax
import jax.numpy as jnp
from jax import lax
from jax.experimental import pallas as pl
from jax.experimental.pallas import tpu as pltpu
from jax.experimental.pallas import tpu_sc as plsc

B = 4
S = 2048
D = 1024
LANES = 16

_info = plsc.get_sparse_core_info()
NC = _info.num_cores
NS = _info.num_subcores
NW = NC * NS

S_PER_W = S // NW
CHUNK = 32
NSUB = S_PER_W // CHUNK
VPR = D // LANES


def _body(ids_hbm, tok_hbm, pos_hbm, out_hbm, idx_v, pos_v, tok_v, sem):
    c = lax.axis_index("c")
    s = lax.axis_index("s")
    wid = s * NC + c
    s0 = wid * S_PER_W

    for sub in range(NSUB):
        base = s0 + sub * CHUNK
        pltpu.sync_copy(pos_hbm.at[pl.ds(base, CHUNK)], pos_v)
        for b in range(B):
            pltpu.sync_copy(ids_hbm.at[pl.ds(b * S + base, CHUNK)], idx_v)
            pltpu.async_copy(tok_hbm.at[idx_v], tok_v, sem).wait()

            def add_body(i, _):
                r = i // VPR
                col = (i % VPR) * LANES
                tok_v[r, pl.ds(col, LANES)] = (
                    tok_v[r, pl.ds(col, LANES)] + pos_v[r, pl.ds(col, LANES)]
                )
                return 0

            lax.fori_loop(0, CHUNK * VPR, add_body, 0)
            pltpu.sync_copy(tok_v, out_hbm.at[b, pl.ds(base, CHUNK)])


_sc_call = functools.partial(
    pl.kernel,
    out_type=jax.ShapeDtypeStruct((B, S, D), jnp.float32),
    mesh=plsc.VectorSubcoreMesh(core_axis_name="c", subcore_axis_name="s"),
    scratch_types=[
        pltpu.VMEM((CHUNK,), jnp.int32),
        pltpu.VMEM((CHUNK, D), jnp.float32),
        pltpu.VMEM((CHUNK, D), jnp.float32),
        pltpu.SemaphoreType.DMA,
    ],
)(_body)


def kernel(input_ids, embed_tokens_weight, embed_positions_weight):
    ids_flat = input_ids.reshape(-1).astype(jnp.int32)
    pos_sliced = lax.slice(embed_positions_weight, (2, 0), (S + 2, D))
    return _sc_call(ids_flat, embed_tokens_weight, pos_sliced)


# speedup vs baseline: 1.8801x; 1.8801x over previous
import functools

import jax
import jax.numpy as jnp
from jax import lax
from jax.experimental import pallas as pl
from jax.experimental.pallas import tpu as pltpu
from jax.experimental.pallas import tpu_sc as plsc

B = 4
S = 2048
D = 1024
LANES = 16

_info = plsc.get_sparse_core_info()
NC = _info.num_cores
NS = _info.num_subcores
NW = NC * NS

S_PER_W = S // NW
CHUNK = 32
NCHUNK = B * (S_PER_W // CHUNK)
VPR = D // LANES


def _body(ids_hbm, tok_hbm, pos_hbm, out_hbm,
          idx_v, pos_v, tok_a, tok_b,
          gsem_a, gsem_b, ssem_a, ssem_b):
    c = lax.axis_index("c")
    s = lax.axis_index("s")
    wid = s * NC + c
    s0 = wid * S_PER_W

    toks = (tok_a, tok_b)
    gsems = (gsem_a, gsem_b)
    ssems = (ssem_a, ssem_b)

    def chunk_coords(k):
        sub, b = divmod(k, B)
        return b, s0 + sub * CHUNK

    def issue_gather(k):
        p = k % 2
        b, base = chunk_coords(k)
        pltpu.sync_copy(ids_hbm.at[pl.ds(b * S + base, CHUNK)], idx_v.at[p])
        return pltpu.async_copy(tok_hbm.at[idx_v.at[p]], toks[p], gsems[p])

    gathers = {}
    stores = {}
    gathers[0] = issue_gather(0)

    for k in range(NCHUNK):
        p = k % 2
        b, base = chunk_coords(k)
        if k + 1 < NCHUNK:
            if k - 1 in stores:
                stores[k - 1].wait()
            gathers[k + 1] = issue_gather(k + 1)
        if b == 0:
            pltpu.sync_copy(pos_hbm.at[pl.ds(base, CHUNK)], pos_v)
        gathers[k].wait()
        tok = toks[p]

        def add_row(r, _):
            for j in range(VPR):
                sl = pl.ds(j * LANES, LANES)
                plsc.addupdate(tok.at[r, sl], pos_v[r, sl])
            return 0

        lax.fori_loop(0, CHUNK, add_row, 0, unroll=False)
        stores[k] = pltpu.async_copy(tok, out_hbm.at[b, pl.ds(base, CHUNK)],
                                     ssems[p])

    stores[NCHUNK - 2].wait()
    stores[NCHUNK - 1].wait()


_sc_call = functools.partial(
    pl.kernel,
    out_type=jax.ShapeDtypeStruct((B, S, D), jnp.float32),
    mesh=plsc.VectorSubcoreMesh(core_axis_name="c", subcore_axis_name="s"),
    scratch_types=[
        pltpu.VMEM((2, CHUNK), jnp.int32),
        pltpu.VMEM((CHUNK, D), jnp.float32),
        pltpu.VMEM((CHUNK, D), jnp.float32),
        pltpu.VMEM((CHUNK, D), jnp.float32),
        pltpu.SemaphoreType.DMA,
        pltpu.SemaphoreType.DMA,
        pltpu.SemaphoreType.DMA,
        pltpu.SemaphoreType.DMA,
    ],
)(_body)


def kernel(input_ids, embed_tokens_weight, embed_positions_weight):
    ids_flat = input_ids.reshape(-1).astype(jnp.int32)
    pos_sliced = lax.slice(embed_positions_weight, (2, 0), (S + 2, D))
    return _sc_call(ids_flat, embed_tokens_weight, pos_sliced)
